# baseline (speedup 1.0000x reference)
import functools

import jax
import jax.numpy as jnp
from jax import lax
from jax.experimental import pallas as pl
from jax.experimental.pallas import tpu as pltpu
from jax.experimental.pallas import tpu_sc as plsc

NC = 2
NS = 16
L = 16
NW = NC * NS
BATCH = 16384
BPW = BATCH // NW
E = 16
CHUNK = 128
NCHUNK = BPW // CHUNK
N_ROWS = 1000000
BC = 32768
C0 = 8 * BC
IPW = C0 // NW
CHD = 256
NCOL = IPW // CHD
NBUFD = 4


def _dot_body(u_ref, i_ref, s_ref):
    s_ref[...] = jnp.sum(u_ref[...] * i_ref[...], axis=0)


@jax.jit
def _dense_dot_tc(ue_t, ie_t):
    nb = pl.cdiv(N_ROWS - C0, BC)
    off = C0 // BC
    return pl.pallas_call(
        _dot_body,
        grid=(nb,),
        in_specs=[
            pl.BlockSpec((E, BC), lambda i: (0, off + i)),
            pl.BlockSpec((E, BC), lambda i: (0, off + i)),
        ],
        out_specs=pl.BlockSpec((BC,), lambda i: (i,)),
        out_shape=jax.ShapeDtypeStruct((N_ROWS - C0, ), jnp.float32),
    )(ue_t, ie_t)


def _dense_body(ue_hbm, ie_hbm, s_hbm, u_v, i_v, s_v, sem):
    wid = lax.axis_index("s") * NC + lax.axis_index("c")
    col0 = wid * NCOL

    def fire(c, slot):
        src = pl.ds((col0 + c) * CHD, CHD)
        for h in range(2):
            rows = pl.ds(8 * h, 8)
            pltpu.async_copy(ue_hbm.at[rows, src], u_v.at[slot, h], sem)
            pltpu.async_copy(ie_hbm.at[rows, src], i_v.at[slot, h], sem)

    for p in range(NBUFD - 1):
        fire(p, p)

    def body(c, carry):
        slot = lax.rem(c, NBUFD)
        nxt = lax.rem(c + NBUFD - 1, NBUFD)

        @pl.when(c + NBUFD - 1 < NCOL)
        def _():
            fire(c + NBUFD - 1, nxt)

        for h in range(2):
            pltpu.make_async_copy(ue_hbm.at[pl.ds(0, 8), pl.ds(0, CHD)],
                                  u_v.at[slot, h], sem).wait()
            pltpu.make_async_copy(ie_hbm.at[pl.ds(0, 8), pl.ds(0, CHD)],
                                  i_v.at[slot, h], sem).wait()

        for j in range(CHD // L):
            d = pl.ds(j * L, L)
            acc = u_v[slot, 0, 0, d] * i_v[slot, 0, 0, d]
            for k in range(1, E):
                acc = acc + u_v[slot, k // 8, k % 8, d] * \
                            i_v[slot, k // 8, k % 8, d]
            s_v[pl.ds(c * CHD + j * L, L)] = acc
        return carry

    lax.fori_loop(0, NCOL, body, 0)
    pltpu.sync_copy(s_v, s_hbm.at[pl.ds(wid * IPW, IPW)])


@functools.cache
def _build_dense_sc():
    mesh = plsc.VectorSubcoreMesh(core_axis_name="c", subcore_axis_name="s",
                                  num_cores=NC, num_subcores=NS)
    return pl.kernel(
        _dense_body,
        out_type=jax.ShapeDtypeStruct((C0,), jnp.float32),
        mesh=mesh,
        scratch_types=[
            pltpu.VMEM((NBUFD, 2, 8, CHD), jnp.float32),
            pltpu.VMEM((NBUFD, 2, 8, CHD), jnp.float32),
            pltpu.VMEM((IPW,), jnp.float32),
            pltpu.SemaphoreType.DMA,
        ],
        compiler_params=pltpu.CompilerParams(needs_layout_passes=False,
                                             use_tc_tiling_on_sc=True),
    )


def _score_body(user_hbm, item_hbm, ub_hbm, ib_hbm, slo_hbm, shi_hbm,
                out_hbm, user_v, item_v, lo_v, hi_v, ub_v, ib_v, slo_v,
                shi_v, out_v, sem):
    wid = lax.axis_index("s") * NC + lax.axis_index("c")

    pltpu.sync_copy(user_hbm.at[pl.ds(wid * NCHUNK, NCHUNK)], user_v)
    pltpu.sync_copy(item_hbm.at[pl.ds(wid * NCHUNK, NCHUNK)], item_v)

    for j in range(NCHUNK):
        for g in range(CHUNK // L):
            d = pl.ds(g * L, L)
            it = item_v[j, d]
            lo_v[j, d] = jnp.minimum(it, C0 - 1)
            hi_v[j, d] = jnp.maximum(it - C0, 0)

    copies = []
    for j in range(NCHUNK):
        dst = pl.ds(j * CHUNK, CHUNK)
        copies.append(pltpu.async_copy(ub_hbm.at[user_v.at[j]],
                                       ub_v.at[dst], sem))
        copies.append(pltpu.async_copy(ib_hbm.at[item_v.at[j]],
                                       ib_v.at[dst], sem))
        copies.append(pltpu.async_copy(slo_hbm.at[lo_v.at[j]],
                                       slo_v.at[dst], sem))
        copies.append(pltpu.async_copy(shi_hbm.at[hi_v.at[j]],
                                       shi_v.at[dst], sem))
    for c in copies:
        c.wait()

    for g in range(BPW // L):
        b = pl.ds(g * L, L)
        it = item_v[g // (CHUNK // L), pl.ds((g % (CHUNK // L)) * L, L)]
        s = jnp.where(it < C0, slo_v[b], shi_v[b])
        out_v[b] = ub_v[b] + ib_v[b] + s

    pltpu.sync_copy(out_v, out_hbm.at[pl.ds(wid * BPW, BPW)])


@functools.cache
def _build_score():
    mesh = plsc.VectorSubcoreMesh(core_axis_name="c", subcore_axis_name="s",
                                  num_cores=NC, num_subcores=NS)
    return pl.kernel(
        _score_body,
        out_type=jax.ShapeDtypeStruct((BATCH,), jnp.float32),
        mesh=mesh,
        scratch_types=[
            pltpu.VMEM((NCHUNK, CHUNK), jnp.int32),
            pltpu.VMEM((NCHUNK, CHUNK), jnp.int32),
            pltpu.VMEM((NCHUNK, CHUNK), jnp.int32),
            pltpu.VMEM((NCHUNK, CHUNK), jnp.int32),
            pltpu.VMEM((BPW,), jnp.float32),
            pltpu.VMEM((BPW,), jnp.float32),
            pltpu.VMEM((BPW,), jnp.float32),
            pltpu.VMEM((BPW,), jnp.float32),
            pltpu.VMEM((BPW,), jnp.float32),
            pltpu.SemaphoreType.DMA,
        ],
        compiler_params=pltpu.CompilerParams(needs_layout_passes=False,
                                             use_tc_tiling_on_sc=False),
    )


def kernel(user, item, u_bias_w, i_bias_w, u_embed_w, i_embed_w):
    user2d = user.astype(jnp.int32).reshape(NW * NCHUNK, CHUNK)
    item2d = item.astype(jnp.int32).reshape(NW * NCHUNK, CHUNK)
    ub = jnp.pad(u_bias_w, ((0, 448), (0, 0))).reshape(-1)
    ib = jnp.pad(i_bias_w, ((0, 448), (0, 0))).reshape(-1)
    ue_t = u_embed_w.T
    ie_t = i_embed_w.T
    s_lo = _build_dense_sc()(ue_t, ie_t)
    s_hi = _dense_dot_tc(ue_t, ie_t)
    return _build_score()(user2d, item2d, ub, ib, s_lo, s_hi)


# speedup vs baseline: 1.6605x; 1.6605x over previous
import functools

import jax
import jax.numpy as jnp
from jax import lax
from jax.experimental import pallas as pl
from jax.experimental.pallas import tpu as pltpu
from jax.experimental.pallas import tpu_sc as plsc

NC = 2
NS = 16
L = 16
NW = NC * NS
BATCH = 16384
BPW = BATCH // NW
E = 16
CHUNK = 128
NCHUNK = BPW // CHUNK
N_ROWS = 1000000
BC = 32768
C0 = 8 * BC
IPW = C0 // NW
CHD = 256
NCOL = IPW // CHD
NBUFD = 4


def _dot_body(u_ref, i_ref, s_ref):
    s_ref[...] = jnp.sum(u_ref[...] * i_ref[...], axis=0)


@jax.jit
def _dense_dot_tc(ue_t, ie_t):
    nb = pl.cdiv(N_ROWS - C0, BC)
    off = C0 // BC
    return pl.pallas_call(
        _dot_body,
        grid=(nb,),
        in_specs=[
            pl.BlockSpec((E, BC), lambda i: (0, off + i)),
            pl.BlockSpec((E, BC), lambda i: (0, off + i)),
        ],
        out_specs=pl.BlockSpec((BC,), lambda i: (i,)),
        out_shape=jax.ShapeDtypeStruct((N_ROWS - C0, ), jnp.float32),
    )(ue_t, ie_t)


def _dense_body(ue_hbm, ie_hbm, s_hbm, u_v, i_v, s_v, sem):
    wid = lax.axis_index("s") * NC + lax.axis_index("c")
    col0 = wid * NCOL

    def fire(c, slot):
        src = pl.ds((col0 + c) * CHD, CHD)
        for h in range(2):
            rows = pl.ds(8 * h, 8)
            pltpu.async_copy(ue_hbm.at[rows, src], u_v.at[slot, h], sem)
            pltpu.async_copy(ie_hbm.at[rows, src], i_v.at[slot, h], sem)

    for p in range(NBUFD - 1):
        fire(p, p)

    def body(c, carry):
        slot = lax.rem(c, NBUFD)
        nxt = lax.rem(c + NBUFD - 1, NBUFD)

        @pl.when(c + NBUFD - 1 < NCOL)
        def _():
            fire(c + NBUFD - 1, nxt)

        for h in range(2):
            pltpu.make_async_copy(ue_hbm.at[pl.ds(0, 8), pl.ds(0, CHD)],
                                  u_v.at[slot, h], sem).wait()
            pltpu.make_async_copy(ie_hbm.at[pl.ds(0, 8), pl.ds(0, CHD)],
                                  i_v.at[slot, h], sem).wait()

        for j in range(CHD // L):
            d = pl.ds(j * L, L)
            acc = u_v[slot, 0, 0, d] * i_v[slot, 0, 0, d]
            for k in range(1, E):
                acc = acc + u_v[slot, k // 8, k % 8, d] * \
                            i_v[slot, k // 8, k % 8, d]
            s_v[pl.ds(c * CHD + j * L, L)] = acc
        return carry

    lax.fori_loop(0, NCOL, body, 0)
    pltpu.sync_copy(s_v, s_hbm.at[pl.ds(wid * IPW, IPW)])


@functools.cache
def _build_dense_sc():
    mesh = plsc.VectorSubcoreMesh(core_axis_name="c", subcore_axis_name="s",
                                  num_cores=NC, num_subcores=NS)
    return pl.kernel(
        _dense_body,
        out_type=jax.ShapeDtypeStruct((C0,), jnp.float32),
        mesh=mesh,
        scratch_types=[
            pltpu.VMEM((NBUFD, 2, 8, CHD), jnp.float32),
            pltpu.VMEM((NBUFD, 2, 8, CHD), jnp.float32),
            pltpu.VMEM((IPW,), jnp.float32),
            pltpu.SemaphoreType.DMA,
        ],
        compiler_params=pltpu.CompilerParams(needs_layout_passes=False,
                                             use_tc_tiling_on_sc=True),
    )


def _score_body(user_hbm, item_hbm, ub_hbm, ib_hbm, s_hbm, out_hbm,
                user_v, item_v, ub_v, ib_v, s_v, out_v, sem):
    wid = lax.axis_index("s") * NC + lax.axis_index("c")

    pltpu.sync_copy(user_hbm.at[pl.ds(wid * NCHUNK, NCHUNK)], user_v)
    pltpu.sync_copy(item_hbm.at[pl.ds(wid * NCHUNK, NCHUNK)], item_v)

    copies = []
    for j in range(NCHUNK):
        dst = pl.ds(j * CHUNK, CHUNK)
        copies.append(pltpu.async_copy(ub_hbm.at[user_v.at[j]],
                                       ub_v.at[dst], sem))
        copies.append(pltpu.async_copy(ib_hbm.at[item_v.at[j]],
                                       ib_v.at[dst], sem))
        copies.append(pltpu.async_copy(s_hbm.at[item_v.at[j]],
                                       s_v.at[dst], sem))
    for c in copies:
        c.wait()

    for g in range(BPW // L):
        b = pl.ds(g * L, L)
        out_v[b] = ub_v[b] + ib_v[b] + s_v[b]

    pltpu.sync_copy(out_v, out_hbm.at[pl.ds(wid * BPW, BPW)])


@functools.cache
def _build_score():
    mesh = plsc.VectorSubcoreMesh(core_axis_name="c", subcore_axis_name="s",
                                  num_cores=NC, num_subcores=NS)
    return pl.kernel(
        _score_body,
        out_type=jax.ShapeDtypeStruct((BATCH,), jnp.float32),
        mesh=mesh,
        scratch_types=[
            pltpu.VMEM((NCHUNK, CHUNK), jnp.int32),
            pltpu.VMEM((NCHUNK, CHUNK), jnp.int32),
            pltpu.VMEM((BPW,), jnp.float32),
            pltpu.VMEM((BPW,), jnp.float32),
            pltpu.VMEM((BPW,), jnp.float32),
            pltpu.VMEM((BPW,), jnp.float32),
            pltpu.SemaphoreType.DMA,
        ],
        compiler_params=pltpu.CompilerParams(needs_layout_passes=False,
                                             use_tc_tiling_on_sc=False),
    )


def kernel(user, item, u_bias_w, i_bias_w, u_embed_w, i_embed_w):
    user2d = user.astype(jnp.int32).reshape(NW * NCHUNK, CHUNK)
    item2d = item.astype(jnp.int32).reshape(NW * NCHUNK, CHUNK)
    ub = jnp.pad(u_bias_w, ((0, 448), (0, 0))).reshape(-1)
    ib = jnp.pad(i_bias_w, ((0, 448), (0, 0))).reshape(-1)
    ue_t = u_embed_w.T
    ie_t = i_embed_w.T
    s_lo = _build_dense_sc()(ue_t, ie_t)
    s_hi = _dense_dot_tc(ue_t, ie_t)
    s = jnp.concatenate([s_lo, s_hi])
    return _build_score()(user2d, item2d, ub, ib, s)


# speedup vs baseline: 1.7130x; 1.0316x over previous
import functools

import jax
import jax.numpy as jnp
from jax import lax
from jax.experimental import pallas as pl
from jax.experimental.pallas import tpu as pltpu
from jax.experimental.pallas import tpu_sc as plsc

NC = 2
NS = 16
L = 16
NW = NC * NS
BATCH = 16384
BPW = BATCH // NW
E = 16
CHUNK = 128
NCHUNK = BPW // CHUNK
N_ROWS = 1000000
BC = 32768
C0 = 8 * BC
IPW = C0 // NW
CHD = 256
NCOL = IPW // CHD
NBUFD = 4


def _dot_body(u_ref, i_ref, s_ref):
    s_ref[...] = jnp.sum(u_ref[...] * i_ref[...], axis=0)


@jax.jit
def _dense_dot_tc(ue_t, ie_t):
    nb = pl.cdiv(N_ROWS - C0, BC)
    off = C0 // BC
    return pl.pallas_call(
        _dot_body,
        grid=(nb,),
        in_specs=[
            pl.BlockSpec((E, BC), lambda i: (0, off + i)),
            pl.BlockSpec((E, BC), lambda i: (0, off + i)),
        ],
        out_specs=pl.BlockSpec((BC,), lambda i: (i,)),
        out_shape=jax.ShapeDtypeStruct((N_ROWS - C0, ), jnp.float32),
    )(ue_t, ie_t)


def _dense_body(ue_hbm, ie_hbm, s_hbm, u_v, i_v, s_v, sem):
    wid = lax.axis_index("s") * NC + lax.axis_index("c")
    col0 = wid * NCOL

    def fire(c, slot):
        src = pl.ds((col0 + c) * CHD, CHD)
        for h in range(2):
            rows = pl.ds(8 * h, 8)
            pltpu.async_copy(ue_hbm.at[rows, src], u_v.at[slot, h], sem)
            pltpu.async_copy(ie_hbm.at[rows, src], i_v.at[slot, h], sem)

    for p in range(NBUFD - 1):
        fire(p, p)

    def body(c, carry):
        slot = lax.rem(c, NBUFD)
        nxt = lax.rem(c + NBUFD - 1, NBUFD)

        @pl.when(c + NBUFD - 1 < NCOL)
        def _():
            fire(c + NBUFD - 1, nxt)

        for h in range(2):
            pltpu.make_async_copy(ue_hbm.at[pl.ds(0, 8), pl.ds(0, CHD)],
                                  u_v.at[slot, h], sem).wait()
            pltpu.make_async_copy(ie_hbm.at[pl.ds(0, 8), pl.ds(0, CHD)],
                                  i_v.at[slot, h], sem).wait()

        for j in range(CHD // L):
            d = pl.ds(j * L, L)
            acc = u_v[slot, 0, 0, d] * i_v[slot, 0, 0, d]
            for k in range(1, E):
                acc = acc + u_v[slot, k // 8, k % 8, d] * \
                            i_v[slot, k // 8, k % 8, d]
            s_v[pl.ds(c * CHD + j * L, L)] = acc
        return carry

    lax.fori_loop(0, NCOL, body, 0)
    pltpu.sync_copy(s_v, s_hbm.at[pl.ds(wid * IPW, IPW)])


@functools.cache
def _build_dense_sc():
    mesh = plsc.VectorSubcoreMesh(core_axis_name="c", subcore_axis_name="s",
                                  num_cores=NC, num_subcores=NS)
    return pl.kernel(
        _dense_body,
        out_type=jax.ShapeDtypeStruct((C0,), jnp.float32),
        mesh=mesh,
        scratch_types=[
            pltpu.VMEM((NBUFD, 2, 8, CHD), jnp.float32),
            pltpu.VMEM((NBUFD, 2, 8, CHD), jnp.float32),
            pltpu.VMEM((IPW,), jnp.float32),
            pltpu.SemaphoreType.DMA,
        ],
        compiler_params=pltpu.CompilerParams(needs_layout_passes=False,
                                             use_tc_tiling_on_sc=True),
    )


def _score_body(user_hbm, item_hbm, ub_hbm, ib_hbm, slo_hbm, shi_hbm,
                out_hbm, user_v, item_v, lo_v, hi_v, ub_v, ib_v, slo_v,
                shi_v, out_v, sem):
    wid = lax.axis_index("s") * NC + lax.axis_index("c")

    pltpu.sync_copy(user_hbm.at[pl.ds(wid * NCHUNK, NCHUNK)], user_v)
    pltpu.sync_copy(item_hbm.at[pl.ds(wid * NCHUNK, NCHUNK)], item_v)

    for j in range(NCHUNK):
        for g in range(CHUNK // L):
            d = pl.ds(g * L, L)
            it = item_v[j, d]
            lo_v[j, d] = jnp.bitwise_and(it, C0 - 1)
            hi_v[j, d] = jnp.where(it >= C0, it - C0,
                                   lax.shift_right_logical(it, 4))

    copies = []
    for j in range(NCHUNK):
        dst = pl.ds(j * CHUNK, CHUNK)
        copies.append(pltpu.async_copy(ub_hbm.at[user_v.at[j]],
                                       ub_v.at[dst], sem))
        copies.append(pltpu.async_copy(ib_hbm.at[item_v.at[j]],
                                       ib_v.at[dst], sem))
        copies.append(pltpu.async_copy(slo_hbm.at[lo_v.at[j]],
                                       slo_v.at[dst], sem))
        copies.append(pltpu.async_copy(shi_hbm.at[hi_v.at[j]],
                                       shi_v.at[dst], sem))
    for c in copies:
        c.wait()

    for g in range(BPW // L):
        b = pl.ds(g * L, L)
        it = item_v[g // (CHUNK // L), pl.ds((g % (CHUNK // L)) * L, L)]
        s = jnp.where(it < C0, slo_v[b], shi_v[b])
        out_v[b] = ub_v[b] + ib_v[b] + s

    pltpu.sync_copy(out_v, out_hbm.at[pl.ds(wid * BPW, BPW)])


@functools.cache
def _build_score():
    mesh = plsc.VectorSubcoreMesh(core_axis_name="c", subcore_axis_name="s",
                                  num_cores=NC, num_subcores=NS)
    return pl.kernel(
        _score_body,
        out_type=jax.ShapeDtypeStruct((BATCH,), jnp.float32),
        mesh=mesh,
        scratch_types=[
            pltpu.VMEM((NCHUNK, CHUNK), jnp.int32),
            pltpu.VMEM((NCHUNK, CHUNK), jnp.int32),
            pltpu.VMEM((NCHUNK, CHUNK), jnp.int32),
            pltpu.VMEM((NCHUNK, CHUNK), jnp.int32),
            pltpu.VMEM((BPW,), jnp.float32),
            pltpu.VMEM((BPW,), jnp.float32),
            pltpu.VMEM((BPW,), jnp.float32),
            pltpu.VMEM((BPW,), jnp.float32),
            pltpu.VMEM((BPW,), jnp.float32),
            pltpu.SemaphoreType.DMA,
        ],
        compiler_params=pltpu.CompilerParams(needs_layout_passes=False,
                                             use_tc_tiling_on_sc=False),
    )


def kernel(user, item, u_bias_w, i_bias_w, u_embed_w, i_embed_w):
    user2d = user.astype(jnp.int32).reshape(NW * NCHUNK, CHUNK)
    item2d = item.astype(jnp.int32).reshape(NW * NCHUNK, CHUNK)
    ub = jnp.pad(u_bias_w, ((0, 448), (0, 0))).reshape(-1)
    ib = jnp.pad(i_bias_w, ((0, 448), (0, 0))).reshape(-1)
    ue_t = u_embed_w.T
    ie_t = i_embed_w.T
    s_lo = _build_dense_sc()(ue_t, ie_t)
    s_hi = _dense_dot_tc(ue_t, ie_t)
    return _build_score()(user2d, item2d, ub, ib, s_lo, s_hi)
